# baseline (speedup 1.0000x reference)
import functools

import jax
import jax.numpy as jnp
from jax import lax
from jax.experimental import pallas as pl
from jax.experimental.pallas import tpu as pltpu
from jax.experimental.pallas import tpu_sc as plsc

EMBED = 16
BATCH = 16384
ALPHA = 0.5

NC = 2
NS = 16
NW = NC * NS
BPW = BATCH // NW
CH = 128
NCHUNK = BPW // CH


def _sc_gather_body(uidx_hbm, iidx_hbm, gu_t, gi_t, mu_t, mi_t,
                    gu_o, gi_o, mu_o, mi_o,
                    uidx_v, iidx_v, gu_v, gi_v, mu_v, mi_v,
                    s0, s1, s2, s3):
    wid = lax.axis_index("s") * NC + lax.axis_index("c")
    base = wid * BPW
    pltpu.sync_copy(uidx_hbm.at[wid], uidx_v)
    pltpu.sync_copy(iidx_hbm.at[wid], iidx_v)
    copies = []
    for j in range(NCHUNK):
        sl = pl.ds(j * CH, CH)
        copies.append(pltpu.async_copy(gu_t.at[uidx_v.at[j]], gu_v.at[sl], s0))
        copies.append(pltpu.async_copy(gi_t.at[iidx_v.at[j]], gi_v.at[sl], s1))
        copies.append(pltpu.async_copy(mu_t.at[uidx_v.at[j]], mu_v.at[sl], s2))
        copies.append(pltpu.async_copy(mi_t.at[iidx_v.at[j]], mi_v.at[sl], s3))
    for c in copies:
        c.wait()
    out_sl = pl.ds(base, BPW)
    pltpu.sync_copy(gu_v, gu_o.at[out_sl])
    pltpu.sync_copy(gi_v, gi_o.at[out_sl])
    pltpu.sync_copy(mu_v, mu_o.at[out_sl])
    pltpu.sync_copy(mi_v, mi_o.at[out_sl])


_sc_gather = functools.partial(
    pl.kernel,
    out_type=[jax.ShapeDtypeStruct((BATCH, EMBED), jnp.float32)] * 4,
    mesh=plsc.VectorSubcoreMesh(core_axis_name="c", subcore_axis_name="s"),
    scratch_types=[
        pltpu.VMEM((NCHUNK, CH), jnp.int32),
        pltpu.VMEM((NCHUNK, CH), jnp.int32),
        pltpu.VMEM((BPW, EMBED), jnp.float32),
        pltpu.VMEM((BPW, EMBED), jnp.float32),
        pltpu.VMEM((BPW, EMBED), jnp.float32),
        pltpu.VMEM((BPW, EMBED), jnp.float32),
        pltpu.SemaphoreType.DMA,
        pltpu.SemaphoreType.DMA,
        pltpu.SemaphoreType.DMA,
        pltpu.SemaphoreType.DMA,
    ],
    compiler_params=pltpu.CompilerParams(use_tc_tiling_on_sc=False),
)(_sc_gather_body)


BLK = 2048


def _tc_mlp_body(gu, gi, mu, mi, w0, b0, w1, b1, w2, b2, w3, b3, wp, bp, out):
    f32 = jnp.float32
    h = jnp.maximum(
        jnp.dot(mu[...], w0[0:EMBED, :], preferred_element_type=f32)
        + jnp.dot(mi[...], w0[EMBED:2 * EMBED, :], preferred_element_type=f32)
        + b0[...], 0.0)
    h = jnp.maximum(jnp.dot(h, w1[...], preferred_element_type=f32) + b1[...], 0.0)
    h = jnp.maximum(jnp.dot(h, w2[...], preferred_element_type=f32) + b2[...], 0.0)
    h = jnp.maximum(jnp.dot(h, w3[...], preferred_element_type=f32) + b3[...], 0.0)
    gmf = gu[...] * gi[...]
    pred = (ALPHA * jnp.dot(gmf, wp[0:EMBED, :], preferred_element_type=f32)
            + (1.0 - ALPHA) * jnp.dot(h, wp[EMBED:, :], preferred_element_type=f32)
            + bp[...])
    out[...] = pred


def _tc_mlp(gu, gi, mu, mi, w0, b0, w1, b1, w2, b2, w3, b3, wp, bp):
    nb = BATCH // BLK
    row_spec = pl.BlockSpec((BLK, EMBED), lambda i: (i, 0))

    def full(a):
        return pl.BlockSpec(a.shape, lambda i: tuple(0 for _ in a.shape))

    return pl.pallas_call(
        _tc_mlp_body,
        grid=(nb,),
        in_specs=[row_spec, row_spec, row_spec, row_spec,
                  full(w0), full(b0), full(w1), full(b1),
                  full(w2), full(b2), full(w3), full(b3),
                  full(wp), full(bp)],
        out_specs=pl.BlockSpec((BLK, 1), lambda i: (i, 0)),
        out_shape=jax.ShapeDtypeStruct((BATCH, 1), jnp.float32),
    )(gu, gi, mu, mi, w0, b0, w1, b1, w2, b2, w3, b3, wp, bp)


def kernel(user_input, item_input, gmf_user_table, gmf_item_table,
           mlp_user_table, mlp_item_table,
           W0, b0, W1, b1, W2, b2, W3, b3, Wp, bp):
    uidx = user_input.astype(jnp.int32).reshape(NW, NCHUNK, CH)
    iidx = item_input.astype(jnp.int32).reshape(NW, NCHUNK, CH)
    gu, gi, mu, mi = _sc_gather(uidx, iidx, gmf_user_table, gmf_item_table,
                                mlp_user_table, mlp_item_table)
    return _tc_mlp(gu, gi, mu, mi,
                   W0, b0.reshape(1, -1), W1, b1.reshape(1, -1),
                   W2, b2.reshape(1, -1), W3, b3.reshape(1, -1),
                   Wp, bp.reshape(1, 1))


# speedup vs baseline: 5.7051x; 5.7051x over previous
import functools

import jax
import jax.numpy as jnp
from jax import lax
from jax.experimental import pallas as pl
from jax.experimental.pallas import tpu as pltpu
from jax.experimental.pallas import tpu_sc as plsc

EMBED = 16
BATCH = 16384
ALPHA = 0.5

NC = 2
NS = 16
NW = NC * NS
BPW = BATCH // NW


RING = 8
NBATCH = BPW // RING
LANE = 128


def _sc_gather_body(uidx_hbm, iidx_hbm, gu_t, gi_t, mu_t, mi_t,
                    gu_o, gi_o, mu_o, mi_o,
                    uidx_s, iidx_s,
                    gu_r, gi_r, mu_r, mi_r,
                    gu_v, gi_v, mu_v, mi_v,
                    s0, s1, s2, s3):
    wid = lax.axis_index("s") * NC + lax.axis_index("c")
    base = wid * BPW
    pltpu.sync_copy(uidx_hbm.at[wid], uidx_s.at[pl.ds(0, BPW)])
    pltpu.sync_copy(iidx_hbm.at[wid], iidx_s.at[pl.ds(0, BPW)])

    tables = ((gu_t, gu_r, s0), (gi_t, gi_r, s1),
              (mu_t, mu_r, s2), (mi_t, mi_r, s3))

    def load_idx(g):
        uvec = uidx_s[pl.ds(g * RING, 16)]
        ivec = iidx_s[pl.ds(g * RING, 16)]
        return uvec, ivec

    def enqueue(g, uvec, ivec):
        for r in range(RING):
            uu = uvec[r]
            ii = ivec[r]
            for tab, ring, sem in tables:
                idx = uu if tab is gu_t or tab is mu_t else ii
                col = idx & jnp.int32(-LANE)
                src = tab.at[:, pl.ds(pl.multiple_of(col, LANE), LANE)]
                pltpu.async_copy(src, ring.at[r], sem)

    enqueue(0, *load_idx(0))
    iota16 = lax.iota(jnp.int32, 16)

    def outer(g, _):
        uvec, ivec = load_idx(g)
        for r in range(RING):
            for tab, ring, sem in tables:
                pltpu.make_async_copy(tab.at[:, pl.ds(0, LANE)], ring.at[r],
                                      sem).wait()

        @pl.when(g + 1 < NBATCH)
        def _():
            enqueue(g + 1, *load_idx(g + 1))

        outs = ((gu_r, uvec, gu_v), (gi_r, ivec, gi_v),
                (mu_r, uvec, mu_v), (mi_r, ivec, mi_v))
        for r in range(RING):
            b = g * RING + r
            for ring, ivecs, out_v in outs:
                u = ivecs[r]
                lane = u & jnp.int32(LANE - 1)
                vec = plsc.load_gather(
                    ring, [jnp.full((16,), r, jnp.int32), iota16,
                           jnp.full((16,), lane, jnp.int32)])
                plsc.store_scatter(out_v,
                                   [iota16, jnp.full((16,), b, jnp.int32)],
                                   vec)
        return ()

    lax.fori_loop(0, NBATCH, outer, ())

    out_sl = pl.ds(base, BPW)
    pltpu.sync_copy(gu_v, gu_o.at[:, out_sl])
    pltpu.sync_copy(gi_v, gi_o.at[:, out_sl])
    pltpu.sync_copy(mu_v, mu_o.at[:, out_sl])
    pltpu.sync_copy(mi_v, mi_o.at[:, out_sl])


_sc_gather = functools.partial(
    pl.kernel,
    out_type=[jax.ShapeDtypeStruct((EMBED, BATCH), jnp.float32)] * 4,
    mesh=plsc.VectorSubcoreMesh(core_axis_name="c", subcore_axis_name="s"),
    scratch_types=[
        pltpu.VMEM((BPW + 16,), jnp.int32),
        pltpu.VMEM((BPW + 16,), jnp.int32),
        pltpu.VMEM((RING, EMBED, LANE), jnp.float32),
        pltpu.VMEM((RING, EMBED, LANE), jnp.float32),
        pltpu.VMEM((RING, EMBED, LANE), jnp.float32),
        pltpu.VMEM((RING, EMBED, LANE), jnp.float32),
        pltpu.VMEM((EMBED, BPW), jnp.float32),
        pltpu.VMEM((EMBED, BPW), jnp.float32),
        pltpu.VMEM((EMBED, BPW), jnp.float32),
        pltpu.VMEM((EMBED, BPW), jnp.float32),
        pltpu.SemaphoreType.DMA,
        pltpu.SemaphoreType.DMA,
        pltpu.SemaphoreType.DMA,
        pltpu.SemaphoreType.DMA,
    ],
    compiler_params=pltpu.CompilerParams(use_tc_tiling_on_sc=True,
                                         needs_layout_passes=False),
)(_sc_gather_body)


BLK = 2048


def _tc_mlp_body(gu, gi, mu, mi, w0, b0, w1, b1, w2, b2, w3, b3, wp, bp, out):
    f32 = jnp.float32
    dims = (((0,), (0,)), ((), ()))
    h = jnp.maximum(
        lax.dot_general(w0[0:EMBED, :], mu[...], dims, preferred_element_type=f32)
        + lax.dot_general(w0[EMBED:2 * EMBED, :], mi[...], dims,
                          preferred_element_type=f32)
        + b0[...], 0.0)
    h = jnp.maximum(
        lax.dot_general(w1[...], h, dims, preferred_element_type=f32) + b1[...], 0.0)
    h = jnp.maximum(
        lax.dot_general(w2[...], h, dims, preferred_element_type=f32) + b2[...], 0.0)
    h = jnp.maximum(
        lax.dot_general(w3[...], h, dims, preferred_element_type=f32) + b3[...], 0.0)
    gmf = gu[...] * gi[...]
    pred = (ALPHA * lax.dot_general(wp[0:EMBED, :], gmf, dims,
                                    preferred_element_type=f32)
            + (1.0 - ALPHA) * lax.dot_general(wp[EMBED:, :], h, dims,
                                              preferred_element_type=f32)
            + bp[...])
    out[...] = pred


def _tc_mlp(gu, gi, mu, mi, w0, b0, w1, b1, w2, b2, w3, b3, wp, bp):
    nb = BATCH // BLK
    col_spec = pl.BlockSpec((EMBED, BLK), lambda i: (0, i))

    def full(a):
        return pl.BlockSpec(a.shape, lambda i: tuple(0 for _ in a.shape))

    return pl.pallas_call(
        _tc_mlp_body,
        grid=(nb,),
        in_specs=[col_spec, col_spec, col_spec, col_spec,
                  full(w0), full(b0), full(w1), full(b1),
                  full(w2), full(b2), full(w3), full(b3),
                  full(wp), full(bp)],
        out_specs=pl.BlockSpec((1, BLK), lambda i: (0, i)),
        out_shape=jax.ShapeDtypeStruct((1, BATCH), jnp.float32),
    )(gu, gi, mu, mi, w0, b0, w1, b1, w2, b2, w3, b3, wp, bp)


def kernel(user_input, item_input, gmf_user_table, gmf_item_table,
           mlp_user_table, mlp_item_table,
           W0, b0, W1, b1, W2, b2, W3, b3, Wp, bp):
    uidx = user_input.astype(jnp.int32).reshape(NW, BPW)
    iidx = item_input.astype(jnp.int32).reshape(NW, BPW)
    gu, gi, mu, mi = _sc_gather(uidx, iidx,
                                gmf_user_table.T, gmf_item_table.T,
                                mlp_user_table.T, mlp_item_table.T)
    pred_t = _tc_mlp(gu, gi, mu, mi,
                     W0, b0.reshape(-1, 1), W1, b1.reshape(-1, 1),
                     W2, b2.reshape(-1, 1), W3, b3.reshape(-1, 1),
                     Wp, bp.reshape(1, 1))
    return pred_t.reshape(BATCH, 1)


# speedup vs baseline: 6.3001x; 1.1043x over previous
import functools

import jax
import jax.numpy as jnp
from jax import lax
from jax.experimental import pallas as pl
from jax.experimental.pallas import tpu as pltpu
from jax.experimental.pallas import tpu_sc as plsc

EMBED = 16
BATCH = 16384
ALPHA = 0.5

NC = 2
NS = 16
NW = NC * NS
BPW = BATCH // NW


RING = 4
NBANK = 2
NBATCH = BPW // RING
NPAIR = NBATCH // NBANK
LANE = 128


def _sc_gather_body(uidx_hbm, iidx_hbm, gu_t, gi_t, mu_t, mi_t,
                    gu_o, gi_o, mu_o, mi_o,
                    uidx_s, iidx_s,
                    gu_r, gi_r, mu_r, mi_r,
                    gu_v, gi_v, mu_v, mi_v,
                    s00, s01, s10, s11, s20, s21, s30, s31):
    wid = lax.axis_index("s") * NC + lax.axis_index("c")
    base = wid * BPW
    pltpu.sync_copy(uidx_hbm.at[wid], uidx_s.at[pl.ds(0, BPW)])
    pltpu.sync_copy(iidx_hbm.at[wid], iidx_s.at[pl.ds(0, BPW)])

    sems = ((s00, s01), (s10, s11), (s20, s21), (s30, s31))
    rings = (gu_r, gi_r, mu_r, mi_r)
    tabs = (gu_t, gi_t, mu_t, mi_t)
    outsv = (gu_v, gi_v, mu_v, mi_v)
    iota16 = lax.iota(jnp.int32, 16)

    def load_idx(gg):
        uvec = uidx_s[pl.ds(gg * NBANK * RING, 16)]
        ivec = iidx_s[pl.ds(gg * NBANK * RING, 16)]
        return uvec, ivec

    def enqueue(bank, uvec, ivec, lane_off):
        for r in range(RING):
            uu = uvec[lane_off + r]
            ii = ivec[lane_off + r]
            for t in range(4):
                idx = uu if t % 2 == 0 else ii
                col = idx & jnp.int32(-LANE)
                src = tabs[t].at[:, pl.ds(pl.multiple_of(col, LANE), LANE)]
                pltpu.async_copy(src, rings[t].at[bank * RING + r],
                                 sems[t][bank])

    def drain(bank):
        for r in range(RING):
            for t in range(4):
                pltpu.make_async_copy(tabs[t].at[:, pl.ds(0, LANE)],
                                      rings[t].at[bank * RING + r],
                                      sems[t][bank]).wait()

    def extract(bank, g, uvec, ivec, lane_off):
        for r in range(RING):
            b = g * RING + r
            for t in range(4):
                u = uvec[lane_off + r] if t % 2 == 0 else ivec[lane_off + r]
                lane = u & jnp.int32(LANE - 1)
                vec = plsc.load_gather(
                    rings[t],
                    [jnp.full((16,), bank * RING + r, jnp.int32), iota16,
                     jnp.full((16,), lane, jnp.int32)])
                plsc.store_scatter(outsv[t],
                                   [iota16, jnp.full((16,), b, jnp.int32)],
                                   vec)

    uvec0, ivec0 = load_idx(0)
    enqueue(0, uvec0, ivec0, 0)

    def outer(gg, _):
        g0 = gg * NBANK
        uvec, ivec = load_idx(gg)
        nvec, jvec = load_idx(gg + 1)
        enqueue(1, uvec, ivec, RING)
        drain(0)
        extract(0, g0, uvec, ivec, 0)

        @pl.when(gg + 1 < NPAIR)
        def _():
            enqueue(0, nvec, jvec, 0)

        drain(1)
        extract(1, g0 + 1, uvec, ivec, RING)
        return ()

    lax.fori_loop(0, NPAIR, outer, ())

    out_sl = pl.ds(base, BPW)
    pltpu.sync_copy(gu_v, gu_o.at[:, out_sl])
    pltpu.sync_copy(gi_v, gi_o.at[:, out_sl])
    pltpu.sync_copy(mu_v, mu_o.at[:, out_sl])
    pltpu.sync_copy(mi_v, mi_o.at[:, out_sl])


_sc_gather = functools.partial(
    pl.kernel,
    out_type=[jax.ShapeDtypeStruct((EMBED, BATCH), jnp.float32)] * 4,
    mesh=plsc.VectorSubcoreMesh(core_axis_name="c", subcore_axis_name="s"),
    scratch_types=[
        pltpu.VMEM((BPW + 16,), jnp.int32),
        pltpu.VMEM((BPW + 16,), jnp.int32),
        pltpu.VMEM((NBANK * RING, EMBED, LANE), jnp.float32),
        pltpu.VMEM((NBANK * RING, EMBED, LANE), jnp.float32),
        pltpu.VMEM((NBANK * RING, EMBED, LANE), jnp.float32),
        pltpu.VMEM((NBANK * RING, EMBED, LANE), jnp.float32),
        pltpu.VMEM((EMBED, BPW), jnp.float32),
        pltpu.VMEM((EMBED, BPW), jnp.float32),
        pltpu.VMEM((EMBED, BPW), jnp.float32),
        pltpu.VMEM((EMBED, BPW), jnp.float32),
        pltpu.SemaphoreType.DMA,
        pltpu.SemaphoreType.DMA,
        pltpu.SemaphoreType.DMA,
        pltpu.SemaphoreType.DMA,
        pltpu.SemaphoreType.DMA,
        pltpu.SemaphoreType.DMA,
        pltpu.SemaphoreType.DMA,
        pltpu.SemaphoreType.DMA,
    ],
    compiler_params=pltpu.CompilerParams(use_tc_tiling_on_sc=True,
                                         needs_layout_passes=False),
)(_sc_gather_body)


BLK = 2048


def _tc_mlp_body(gu, gi, mu, mi, w0, b0, w1, b1, w2, b2, w3, b3, wp, bp, out):
    f32 = jnp.float32
    dims = (((0,), (0,)), ((), ()))
    h = jnp.maximum(
        lax.dot_general(w0[0:EMBED, :], mu[...], dims, preferred_element_type=f32)
        + lax.dot_general(w0[EMBED:2 * EMBED, :], mi[...], dims,
                          preferred_element_type=f32)
        + b0[...], 0.0)
    h = jnp.maximum(
        lax.dot_general(w1[...], h, dims, preferred_element_type=f32) + b1[...], 0.0)
    h = jnp.maximum(
        lax.dot_general(w2[...], h, dims, preferred_element_type=f32) + b2[...], 0.0)
    h = jnp.maximum(
        lax.dot_general(w3[...], h, dims, preferred_element_type=f32) + b3[...], 0.0)
    gmf = gu[...] * gi[...]
    pred = (ALPHA * lax.dot_general(wp[0:EMBED, :], gmf, dims,
                                    preferred_element_type=f32)
            + (1.0 - ALPHA) * lax.dot_general(wp[EMBED:, :], h, dims,
                                              preferred_element_type=f32)
            + bp[...])
    out[...] = pred


def _tc_mlp(gu, gi, mu, mi, w0, b0, w1, b1, w2, b2, w3, b3, wp, bp):
    nb = BATCH // BLK
    col_spec = pl.BlockSpec((EMBED, BLK), lambda i: (0, i))

    def full(a):
        return pl.BlockSpec(a.shape, lambda i: tuple(0 for _ in a.shape))

    return pl.pallas_call(
        _tc_mlp_body,
        grid=(nb,),
        in_specs=[col_spec, col_spec, col_spec, col_spec,
                  full(w0), full(b0), full(w1), full(b1),
                  full(w2), full(b2), full(w3), full(b3),
                  full(wp), full(bp)],
        out_specs=pl.BlockSpec((1, BLK), lambda i: (0, i)),
        out_shape=jax.ShapeDtypeStruct((1, BATCH), jnp.float32),
    )(gu, gi, mu, mi, w0, b0, w1, b1, w2, b2, w3, b3, wp, bp)


def kernel(user_input, item_input, gmf_user_table, gmf_item_table,
           mlp_user_table, mlp_item_table,
           W0, b0, W1, b1, W2, b2, W3, b3, Wp, bp):
    uidx = user_input.astype(jnp.int32).reshape(NW, BPW)
    iidx = item_input.astype(jnp.int32).reshape(NW, BPW)
    gu, gi, mu, mi = _sc_gather(uidx, iidx,
                                gmf_user_table.T, gmf_item_table.T,
                                mlp_user_table.T, mlp_item_table.T)
    pred_t = _tc_mlp(gu, gi, mu, mi,
                     W0, b0.reshape(-1, 1), W1, b1.reshape(-1, 1),
                     W2, b2.reshape(-1, 1), W3, b3.reshape(-1, 1),
                     Wp, bp.reshape(1, 1))
    return pred_t.reshape(BATCH, 1)
